# Initial kernel scaffold; baseline (speedup 1.0000x reference)
import jax
import jax.numpy as jnp
from jax.experimental import pallas as pl


def kernel(embed_weight, y_context, indices):
    raise NotImplementedError("write your pallas kernel here")


# speedup vs baseline: 2.2286x; 2.2286x over previous
import functools

import jax
import jax.numpy as jnp
from jax import lax
from jax.experimental import pallas as pl
from jax.experimental.pallas import tpu as pltpu
from jax.experimental.pallas import tpu_sc as plsc

K_NN = 10
TAU = 1.0

NC, NS, LANES = 2, 16, 16
NW = NC * NS


def _topk_body(wp_ref, wpt_ref, nbr_ref, wts_ref, *, v_real):
    wp = wp_ref[...]
    wpt = wpt_ref[...]
    g = jnp.dot(wp, wpt, preferred_element_type=jnp.float32)
    n_row = jnp.sum(wp * wp, axis=1, keepdims=True)
    n_col = jnp.sum(wpt * wpt, axis=0, keepdims=True)
    d2 = n_row - 2.0 * g + n_col

    vp = d2.shape[0]
    col = lax.broadcasted_iota(jnp.int32, (vp, vp), 1)
    row = lax.broadcasted_iota(jnp.int32, (vp, vp), 0)
    inf = jnp.float32(3e38)
    d2m = jnp.where((col == row) | (col >= v_real), inf, d2)

    ids = []
    ws = []
    for _ in range(K_NN):
        m = jnp.min(d2m, axis=1, keepdims=True)
        hit = d2m == m
        idx = jnp.min(jnp.where(hit, col, vp), axis=1, keepdims=True)
        ids.append(idx)
        w = jnp.exp(-(jnp.sqrt(jnp.maximum(m, 1e-12)) + 0.001) / TAU)
        ws.append(w)
        d2m = jnp.where(col == idx, inf, d2m)

    pad_i = jnp.zeros((vp, 1), jnp.int32)
    pad_w = jnp.zeros((vp, 1), jnp.float32)
    nbr_ref[...] = jnp.concatenate(ids + [pad_i] * (16 - K_NN), axis=1)
    wts_ref[...] = jnp.concatenate(ws + [pad_w] * (16 - K_NN), axis=1)


def _topk_call(wp, wpt, v_real):
    vp = wp.shape[0]
    return pl.pallas_call(
        functools.partial(_topk_body, v_real=v_real),
        out_shape=(
            jax.ShapeDtypeStruct((vp, 16), jnp.int32),
            jax.ShapeDtypeStruct((vp, 16), jnp.float32),
        ),
    )(wp, wpt)


def _sc_gather_body(idx_hbm, nbr_hbm, wts_hbm, y_hbm, s_out_hbm, wb_out_hbm,
                    idxv, nbrr, wtr, fidx, sdata, sem, *, bpw, t_len, v_real):
    wid = lax.axis_index("s") * NC + lax.axis_index("c")
    base = wid * bpw

    pltpu.sync_copy(idx_hbm.at[pl.ds(base, bpw)], idxv)
    pltpu.async_copy(nbr_hbm.at[idxv], nbrr, sem).wait()
    pltpu.async_copy(wts_hbm.at[idxv], wtr, sem).wait()
    pltpu.sync_copy(wtr, wb_out_hbm.at[pl.ds(base, bpw)])

    def expand(bb, carry):
        nv = nbrr[bb]
        rowbase = (base + bb) * (t_len * v_real)
        def tstep(t, c):
            fidx[pl.ds((bb * t_len + t) * LANES, LANES)] = nv + (rowbase + t * v_real)
            return c
        return lax.fori_loop(0, t_len, tstep, carry, unroll=True)
    lax.fori_loop(0, bpw, expand, 0)

    total = bpw * t_len * LANES
    nchunk = total // 128
    group = 8
    def gather_group(gidx, carry):
        copies = []
        for i in range(group):
            c = gidx * group + i
            cp = pltpu.make_async_copy(
                y_hbm.at[fidx.at[pl.ds(c * 128, 128)]],
                sdata.at[pl.ds(c * 128, 128)],
                sem,
            )
            cp.start()
            copies.append(cp)
        for cp in copies:
            cp.wait()
        return carry
    lax.fori_loop(0, nchunk // group, gather_group, 0)

    pltpu.sync_copy(sdata, s_out_hbm.at[pl.ds(base * t_len * LANES, total)])


def _sc_gather_call(idx32, nbr, wts, y_flat, b, t_len, v_real):
    bpw = b // NW
    mesh = plsc.VectorSubcoreMesh(core_axis_name="c", subcore_axis_name="s")
    kern = pl.kernel(
        functools.partial(_sc_gather_body, bpw=bpw, t_len=t_len, v_real=v_real),
        out_type=(
            jax.ShapeDtypeStruct((b * t_len * LANES,), jnp.float32),
            jax.ShapeDtypeStruct((b, 16), jnp.float32),
        ),
        mesh=mesh,
        compiler_params=pltpu.CompilerParams(use_tc_tiling_on_sc=False),
        scratch_types=(
            pltpu.VMEM((bpw,), jnp.int32),
            pltpu.VMEM((bpw, 16), jnp.int32),
            pltpu.VMEM((bpw, 16), jnp.float32),
            pltpu.VMEM((bpw * t_len * LANES,), jnp.int32),
            pltpu.VMEM((bpw * t_len * LANES,), jnp.float32),
            pltpu.SemaphoreType.DMA,
        ),
    )
    return kern(idx32, nbr, wts, y_flat)


def _combine_body(s_ref, w_ref, out_ref):
    s = s_ref[...]
    w = w_ref[...][:, None, :]
    lane = lax.broadcasted_iota(jnp.int32, s.shape, 2)
    mask = lane < K_NN
    tw = jnp.where(s == 0.0, w / 1e9, w)
    twm = jnp.where(mask, tw, 0.0)
    f1 = jnp.sum(twm, axis=2, keepdims=True)
    f2 = jnp.sum(jnp.where(mask, tw * s, 0.0), axis=2, keepdims=True) / f1
    sm = jnp.where(mask, s, 0.0)
    mean = jnp.sum(sm, axis=2, keepdims=True) / K_NN
    var = jnp.sum(jnp.where(mask, (s - mean) ** 2, 0.0), axis=2, keepdims=True) / (K_NN - 1)
    f3 = jnp.sqrt(var)
    out_ref[...] = jnp.concatenate([f1, f2, f3], axis=2)


def _combine_call(s3, wb, b, t_len):
    bb = 128
    grid = b // bb
    return pl.pallas_call(
        _combine_body,
        grid=(grid,),
        in_specs=[
            pl.BlockSpec((bb, t_len, 16), lambda i: (i, 0, 0)),
            pl.BlockSpec((bb, 16), lambda i: (i, 0)),
        ],
        out_specs=pl.BlockSpec((bb, t_len, 3), lambda i: (i, 0, 0)),
        out_shape=jax.ShapeDtypeStruct((b, t_len, 3), jnp.float32),
    )(s3, wb)


def kernel(embed_weight, y_context, indices):
    v_real, d = embed_weight.shape
    b, t_len, _ = y_context.shape
    vp = 1024

    wp = jnp.zeros((vp, 128), jnp.float32).at[:v_real, :d].set(embed_weight)
    wpt = wp.T

    nbr, wts = _topk_call(wp, wpt, v_real)

    idx32 = indices.astype(jnp.int32)
    y_flat = y_context.reshape(-1)
    s_flat, wb = _sc_gather_call(idx32, nbr, wts, y_flat, b, t_len, v_real)
    s3 = s_flat.reshape(b, t_len, 16)
    return _combine_call(s3, wb, b, t_len)


# speedup vs baseline: 2.4523x; 1.1004x over previous
import functools

import jax
import jax.numpy as jnp
from jax import lax
from jax.experimental import pallas as pl
from jax.experimental.pallas import tpu as pltpu
from jax.experimental.pallas import tpu_sc as plsc

K_NN = 10
TAU = 1.0

NC, NS, LANES = 2, 16, 16
NW = NC * NS


def _topk_body(wp_ref, wpt_ref, nbr_ref, wts_ref, *, v_real):
    wp = wp_ref[...]
    wpt = wpt_ref[...]
    g = jnp.dot(wp, wpt, preferred_element_type=jnp.float32)
    n_row = jnp.sum(wp * wp, axis=1, keepdims=True)
    n_col = jnp.sum(wpt * wpt, axis=0, keepdims=True)
    d2 = n_row - 2.0 * g + n_col

    vp = d2.shape[0]
    col = lax.broadcasted_iota(jnp.int32, (vp, vp), 1)
    row = lax.broadcasted_iota(jnp.int32, (vp, vp), 0)
    inf = jnp.float32(3e38)
    d2m = jnp.where((col == row) | (col >= v_real), inf, d2)

    ids = []
    ws = []
    for _ in range(K_NN):
        m = jnp.min(d2m, axis=1, keepdims=True)
        hit = d2m == m
        idx = jnp.min(jnp.where(hit, col, vp), axis=1, keepdims=True)
        ids.append(idx)
        w = jnp.exp(-(jnp.sqrt(jnp.maximum(m, 1e-12)) + 0.001) / TAU)
        ws.append(w)
        d2m = jnp.where(col == idx, inf, d2m)

    pad_i = jnp.zeros((vp, 1), jnp.int32)
    pad_w = jnp.zeros((vp, 1), jnp.float32)
    nbr_ref[...] = jnp.concatenate(ids + [pad_i] * (16 - K_NN), axis=1)
    wts_ref[...] = jnp.concatenate(ws + [pad_w] * (16 - K_NN), axis=1)


def _topk_call(wp, wpt, v_real):
    vp = wp.shape[0]
    return pl.pallas_call(
        functools.partial(_topk_body, v_real=v_real),
        out_shape=(
            jax.ShapeDtypeStruct((vp, 16), jnp.int32),
            jax.ShapeDtypeStruct((vp, 16), jnp.float32),
        ),
    )(wp, wpt)


def _sc_gather_body(idx_hbm, nbr_hbm, wts_hbm, y_hbm, s_out_hbm, wb_out_hbm,
                    idxv, nbrr, wtr, nbrt, wbt, fidx, sdata, sem,
                    *, bpw, t_len, v_real):
    wid = lax.axis_index("s") * NC + lax.axis_index("c")
    base = wid * bpw
    bgs = bpw // LANES

    pltpu.sync_copy(idx_hbm.at[pl.ds(base, bpw)], idxv)
    pltpu.async_copy(nbr_hbm.at[idxv], nbrr, sem).wait()
    pltpu.async_copy(wts_hbm.at[idxv], wtr, sem).wait()

    lane = lax.broadcasted_iota(jnp.int32, (LANES,), 0)
    for j in range(K_NN):
        jvec = jnp.full((LANES,), j, jnp.int32)
        for bg in range(bgs):
            rows = bg * LANES + lane
            nbrt[j, pl.ds(bg * LANES, LANES)] = plsc.load_gather(nbrr, [rows, jvec])
            wbt[j, pl.ds(bg * LANES, LANES)] = plsc.load_gather(wtr, [rows, jvec])
    pltpu.sync_copy(wbt, wb_out_hbm.at[wid])

    for bg in range(bgs):
        bvec = (base + bg * LANES + lane) * (t_len * v_real)
        def jbody(j, carry, bg=bg, bvec=bvec):
            njb = nbrt[j, pl.ds(bg * LANES, LANES)] + bvec
            def tbody(t, c, j=j, bg=bg, njb=njb):
                fidx[pl.ds(((j * t_len + t) * bgs + bg) * LANES, LANES)] = (
                    njb + t * v_real)
                return c
            return lax.fori_loop(0, t_len, tbody, carry, unroll=True)
        lax.fori_loop(0, K_NN, jbody, 0)

    total = bpw * t_len * K_NN
    nchunk = total // 128
    group = 8
    def gather_group(gidx, carry):
        copies = []
        for i in range(group):
            c = gidx * group + i
            cp = pltpu.make_async_copy(
                y_hbm.at[fidx.at[pl.ds(c * 128, 128)]],
                sdata.at[c],
                sem,
            )
            cp.start()
            copies.append(cp)
        for cp in copies:
            cp.wait()
        return carry
    lax.fori_loop(0, nchunk // group, gather_group, 0)

    pltpu.sync_copy(sdata, s_out_hbm.at[pl.ds(wid * K_NN * t_len, K_NN * t_len)])


def _sc_gather_call(idx32, nbr, wts, y_flat, b, t_len, v_real):
    bpw = b // NW
    nrows = K_NN * t_len
    mesh = plsc.VectorSubcoreMesh(core_axis_name="c", subcore_axis_name="s")
    kern = pl.kernel(
        functools.partial(_sc_gather_body, bpw=bpw, t_len=t_len, v_real=v_real),
        out_type=(
            jax.ShapeDtypeStruct((NW * nrows, bpw), jnp.float32),
            jax.ShapeDtypeStruct((NW, K_NN, bpw), jnp.float32),
        ),
        mesh=mesh,
        compiler_params=pltpu.CompilerParams(
            use_tc_tiling_on_sc=False, needs_layout_passes=False),
        scratch_types=(
            pltpu.VMEM((bpw,), jnp.int32),
            pltpu.VMEM((bpw, 16), jnp.int32),
            pltpu.VMEM((bpw, 16), jnp.float32),
            pltpu.VMEM((K_NN, bpw), jnp.int32),
            pltpu.VMEM((K_NN, bpw), jnp.float32),
            pltpu.VMEM((nrows * bpw,), jnp.int32),
            pltpu.VMEM((nrows, bpw), jnp.float32),
            pltpu.SemaphoreType.DMA,
        ),
    )
    return kern(idx32, nbr, wts, y_flat)


def _combine_body(s_ref, w_ref, out_ref, *, t_len):
    s = s_ref[...]
    s3 = s.reshape(K_NN, t_len, s.shape[-1])
    w3 = w_ref[0][:, None, :]
    tw = jnp.where(s3 == 0.0, w3 / 1e9, w3)
    f1 = jnp.sum(tw, axis=0)
    f2 = jnp.sum(tw * s3, axis=0) / f1
    mean = jnp.sum(s3, axis=0) / K_NN
    var = jnp.sum((s3 - mean) ** 2, axis=0) / (K_NN - 1)
    f3 = jnp.sqrt(var)
    out_ref[...] = jnp.stack([f1.T, f2.T, f3.T], axis=2)


def _combine_call(s_out, wb, b, t_len):
    bpw = b // NW
    nrows = K_NN * t_len
    return pl.pallas_call(
        functools.partial(_combine_body, t_len=t_len),
        grid=(NW,),
        in_specs=[
            pl.BlockSpec((nrows, bpw), lambda i: (i, 0)),
            pl.BlockSpec((1, K_NN, bpw), lambda i: (i, 0, 0)),
        ],
        out_specs=pl.BlockSpec((bpw, t_len, 3), lambda i: (i, 0, 0)),
        out_shape=jax.ShapeDtypeStruct((b, t_len, 3), jnp.float32),
    )(s_out, wb)


def kernel(embed_weight, y_context, indices):
    v_real, d = embed_weight.shape
    b, t_len, _ = y_context.shape
    vp = 1024

    wp = jnp.zeros((vp, 128), jnp.float32).at[:v_real, :d].set(embed_weight)
    wpt = wp.T

    nbr, wts = _topk_call(wp, wpt, v_real)

    idx32 = indices.astype(jnp.int32)
    y_flat = y_context.reshape(-1)
    s_out, wb = _sc_gather_call(idx32, nbr, wts, y_flat, b, t_len, v_real)
    return _combine_call(s_out, wb, b, t_len)


# speedup vs baseline: 2.7941x; 1.1394x over previous
import functools

import jax
import jax.numpy as jnp
from jax import lax
from jax.experimental import pallas as pl
from jax.experimental.pallas import tpu as pltpu
from jax.experimental.pallas import tpu_sc as plsc

K_NN = 10
TAU = 1.0

NC, NS, LANES = 2, 16, 16
NW = NC * NS


def _topk_body(wp_ref, wpt_ref, nbrw_ref, *, v_real):
    wp = wp_ref[...]
    wpt = wpt_ref[...]
    g = jnp.dot(wp, wpt, preferred_element_type=jnp.float32)
    n_row = jnp.sum(wp * wp, axis=1, keepdims=True)
    n_col = jnp.sum(wpt * wpt, axis=0, keepdims=True)
    d2 = n_row - 2.0 * g + n_col

    vp = d2.shape[0]
    col = lax.broadcasted_iota(jnp.int32, (vp, vp), 1)
    row = lax.broadcasted_iota(jnp.int32, (vp, vp), 0)
    inf = jnp.float32(3e38)
    d2m = jnp.where((col == row) | (col >= v_real), inf, d2)

    ids = []
    wbits = []
    for _ in range(K_NN):
        m = jnp.min(d2m, axis=1, keepdims=True)
        hit = d2m == m
        idx = jnp.min(jnp.where(hit, col, vp), axis=1, keepdims=True)
        ids.append(idx)
        w = jnp.exp(-(jnp.sqrt(jnp.maximum(m, 1e-12)) + 0.001) / TAU)
        wbits.append(lax.bitcast_convert_type(w, jnp.int32))
        d2m = jnp.where(col == idx, inf, d2m)

    zed = jnp.zeros((vp, 1), jnp.int32)
    cols = ids + [zed] * (16 - K_NN) + wbits + [zed] * (112 - K_NN)
    nbrw_ref[...] = jnp.concatenate(cols, axis=1)


def _topk_call(wp, wpt, v_real):
    vp = wp.shape[0]
    return pl.pallas_call(
        functools.partial(_topk_body, v_real=v_real),
        out_shape=jax.ShapeDtypeStruct((vp, 128), jnp.int32),
    )(wp, wpt)


def _sc_gather_body(idx_hbm, nbrw_hbm, y_hbm, s_out_hbm, wb_out_hbm,
                    idxv, nwr, nbrt, wbt, plane_a, plane_b, sdata,
                    sem, psem_a, psem_b, *, bpw, t_len, v_real):
    wid = lax.axis_index("s") * NC + lax.axis_index("c")
    base = wid * bpw
    bgs = bpw // LANES
    lane = lax.broadcasted_iota(jnp.int32, (LANES,), 0)

    pltpu.sync_copy(idx_hbm.at[pl.ds(base, bpw)], idxv)
    pltpu.async_copy(nbrw_hbm.at[idxv], nwr, sem).wait()

    for j in range(K_NN):
        jid = jnp.full((LANES,), j, jnp.int32)
        jwt = jnp.full((LANES,), 16 + j, jnp.int32)
        for bg in range(bgs):
            rows = bg * LANES + lane
            nbrt[j, pl.ds(bg * LANES, LANES)] = plsc.load_gather(nwr, [rows, jid])
            wbt[j, pl.ds(bg * LANES, LANES)] = plsc.load_gather(nwr, [rows, jwt])
    pltpu.sync_copy(wbt, wb_out_hbm.at[wid])

    def extract(bb, plane):
        bvec = jnp.full((LANES,), bb, jnp.int32)
        t_lo = lane
        t_hi = 16 + (lane & 3)
        himask = lane < (t_len - LANES)
        for j in range(K_NN):
            jv = jnp.full((LANES,), j, jnp.int32)
            vsplat = plsc.load_gather(nbrt, [jv, bvec])
            e_lo = plsc.load_gather(plane, [t_lo, vsplat])
            e_hi = plsc.load_gather(plane, [t_hi, vsplat])
            plsc.store_scatter(sdata, [j * t_len + t_lo, bvec], e_lo)
            plsc.store_scatter(sdata, [j * t_len + t_hi, bvec], e_hi, mask=himask)

    cp_a = pltpu.make_async_copy(y_hbm.at[base], plane_a, psem_a)
    cp_a.start()
    def pair(g, carry):
        b0 = 2 * g
        cp_b = pltpu.make_async_copy(y_hbm.at[base + b0 + 1], plane_b, psem_b)
        cp_b.start()
        pltpu.make_async_copy(y_hbm.at[base + b0], plane_a, psem_a).wait()
        extract(b0, plane_a)
        @pl.when(b0 + 2 < bpw)
        def _():
            pltpu.make_async_copy(y_hbm.at[base + b0 + 2], plane_a, psem_a).start()
        cp_b.wait()
        extract(b0 + 1, plane_b)
        return carry
    lax.fori_loop(0, bpw // 2, pair, 0)

    pltpu.sync_copy(sdata, s_out_hbm.at[pl.ds(wid * K_NN * t_len, K_NN * t_len)])


def _sc_gather_call(idx32, nbrw, y3, b, t_len, v_real):
    bpw = b // NW
    nrows = K_NN * t_len
    mesh = plsc.VectorSubcoreMesh(core_axis_name="c", subcore_axis_name="s")
    kern = pl.kernel(
        functools.partial(_sc_gather_body, bpw=bpw, t_len=t_len, v_real=v_real),
        out_type=(
            jax.ShapeDtypeStruct((NW * nrows, bpw), jnp.float32),
            jax.ShapeDtypeStruct((NW, 16, bpw), jnp.int32),
        ),
        mesh=mesh,
        compiler_params=pltpu.CompilerParams(
            use_tc_tiling_on_sc=True, needs_layout_passes=False),
        scratch_types=(
            pltpu.VMEM((bpw,), jnp.int32),
            pltpu.VMEM((bpw, 128), jnp.int32),
            pltpu.VMEM((16, bpw), jnp.int32),
            pltpu.VMEM((16, bpw), jnp.int32),
            pltpu.VMEM((t_len, v_real), jnp.float32),
            pltpu.VMEM((t_len, v_real), jnp.float32),
            pltpu.VMEM((nrows, bpw), jnp.float32),
            pltpu.SemaphoreType.DMA,
            pltpu.SemaphoreType.DMA,
            pltpu.SemaphoreType.DMA,
        ),
    )
    return kern(idx32, nbrw, y3)


def _combine_body(s_ref, w_ref, out_ref, *, t_len):
    s = s_ref[...]
    s3 = s.reshape(K_NN, t_len, s.shape[-1])
    wf = lax.bitcast_convert_type(w_ref[0], jnp.float32)
    w3 = wf[:K_NN][:, None, :]
    tw = jnp.where(s3 == 0.0, w3 / 1e9, w3)
    f1 = jnp.sum(tw, axis=0)
    f2 = jnp.sum(tw * s3, axis=0) / f1
    mean = jnp.sum(s3, axis=0) / K_NN
    var = jnp.sum((s3 - mean) ** 2, axis=0) / (K_NN - 1)
    f3 = jnp.sqrt(var)
    out_ref[...] = jnp.stack([f1.T, f2.T, f3.T], axis=2)


def _combine_call(s_out, wb, b, t_len):
    bpw = b // NW
    nrows = K_NN * t_len
    return pl.pallas_call(
        functools.partial(_combine_body, t_len=t_len),
        grid=(NW,),
        in_specs=[
            pl.BlockSpec((nrows, bpw), lambda i: (i, 0)),
            pl.BlockSpec((1, 16, bpw), lambda i: (i, 0, 0)),
        ],
        out_specs=pl.BlockSpec((bpw, t_len, 3), lambda i: (i, 0, 0)),
        out_shape=jax.ShapeDtypeStruct((b, t_len, 3), jnp.float32),
    )(s_out, wb)


def kernel(embed_weight, y_context, indices):
    v_real, d = embed_weight.shape
    b, t_len, _ = y_context.shape
    vp = 1024

    wp = jnp.zeros((vp, 128), jnp.float32).at[:v_real, :d].set(embed_weight)
    wpt = wp.T

    nbrw = _topk_call(wp, wpt, v_real)

    idx32 = indices.astype(jnp.int32)
    s_out, wb = _sc_gather_call(idx32, nbrw, y_context, b, t_len, v_real)
    return _combine_call(s_out, wb, b, t_len)


# speedup vs baseline: 8.5980x; 3.0772x over previous
import functools

import jax
import jax.numpy as jnp
from jax import lax
from jax.experimental import pallas as pl
from jax.experimental.pallas import tpu as pltpu
from jax.experimental.pallas import tpu_sc as plsc

K_NN = 10
TAU = 1.0

NC, NS, LANES = 2, 16, 16
NW = NC * NS
NQ = 5


def _topk_body(wp_ref, wpt_ref, nbrw_ref, *, v_real):
    wp = wp_ref[...]
    wpt = wpt_ref[...]
    g = jnp.dot(wp, wpt, preferred_element_type=jnp.float32)
    n_row = jnp.sum(wp * wp, axis=1, keepdims=True)
    n_col = jnp.sum(wpt * wpt, axis=0, keepdims=True)
    d2 = n_row - 2.0 * g + n_col

    vp = d2.shape[0]
    col = lax.broadcasted_iota(jnp.int32, (vp, vp), 1)
    row = lax.broadcasted_iota(jnp.int32, (vp, vp), 0)
    inf = jnp.float32(3e38)
    d2m = jnp.where((col == row) | (col >= v_real), inf, d2)

    ids = []
    wbits = []
    for _ in range(K_NN):
        m = jnp.min(d2m, axis=1, keepdims=True)
        hit = d2m == m
        idx = jnp.min(jnp.where(hit, col, vp), axis=1, keepdims=True)
        ids.append(idx)
        w = jnp.exp(-(jnp.sqrt(jnp.maximum(m, 1e-12)) + 0.001) / TAU)
        wbits.append(lax.bitcast_convert_type(w, jnp.int32))
        d2m = jnp.where(col == idx, inf, d2m)

    zed = jnp.zeros((vp, 1), jnp.int32)
    cols = ids + [zed] * (16 - K_NN) + wbits + [zed] * (112 - K_NN)
    nbrw_ref[...] = jnp.concatenate(cols, axis=1)


def _topk_call(wp, wpt, v_real):
    vp = wp.shape[0]
    return pl.pallas_call(
        functools.partial(_topk_body, v_real=v_real),
        out_shape=jax.ShapeDtypeStruct((vp, 128), jnp.int32),
    )(wp, wpt)


def _sc_gather_body(idx_hbm, nbrw_hbm, yt_hbm, s_out_hbm, wb_out_hbm,
                    idxv, nwr, nbrt, wbt, vloc_pre, mask_pre,
                    slab_a, slab_b, sdata, sem, psem_a, psem_b,
                    *, bpw, t_len, v_real, vq):
    wid = lax.axis_index("s") * NC + lax.axis_index("c")
    base = wid * bpw
    bgs = bpw // LANES
    lane = lax.broadcasted_iota(jnp.int32, (LANES,), 0)

    pltpu.sync_copy(idx_hbm.at[pl.ds(base, bpw)], idxv)
    pltpu.async_copy(nbrw_hbm.at[idxv], nwr, sem).wait()

    for j in range(K_NN):
        jid = jnp.full((LANES,), j, jnp.int32)
        jwt = jnp.full((LANES,), 16 + j, jnp.int32)
        for bg in range(bgs):
            rows = bg * LANES + lane
            nbrt[j, pl.ds(bg * LANES, LANES)] = plsc.load_gather(nwr, [rows, jid])
            wbt[j, pl.ds(bg * LANES, LANES)] = plsc.load_gather(nwr, [rows, jwt])
    pltpu.sync_copy(wbt, wb_out_hbm.at[wid])

    for j in range(K_NN):
        for bg in range(bgs):
            sl = pl.ds(bg * LANES, LANES)
            v = nbrt[j, sl]
            for h in range(NQ):
                vl = jnp.clip(v - h * vq, 0, vq - 1)
                inr = (v >= h * vq) & (v < (h + 1) * vq)
                vloc_pre[h * K_NN + j, sl] = vl
                mask_pre[h * K_NN + j, sl] = jnp.where(inr, 1, 0)

    nsteps = t_len * NQ

    def slab_copy(g, slab, psem):
        t = g // NQ
        h = g - t * NQ
        return pltpu.make_async_copy(
            yt_hbm.at[t, pl.ds(h * vq, vq), pl.ds(base, bpw)], slab, psem)

    def extract(g, slab):
        t = g // NQ
        h = g - t * NQ
        for j in range(K_NN):
            rowv = jnp.full((LANES,), j * t_len + t, jnp.int32)
            for bg in range(bgs):
                sl = pl.ds(bg * LANES, LANES)
                vl = vloc_pre[h * K_NN + j, sl]
                mk = mask_pre[h * K_NN + j, sl] > 0
                bvec = bg * LANES + lane
                e = plsc.load_gather(slab, [vl, bvec])
                plsc.store_scatter(sdata, [rowv, bvec], e, mask=mk)

    slab_copy(0, slab_a, psem_a).start()

    def pair(p, carry):
        g0 = 2 * p
        cp_b = slab_copy(g0 + 1, slab_b, psem_b)
        cp_b.start()
        slab_copy(g0, slab_a, psem_a).wait()
        extract(g0, slab_a)
        @pl.when(g0 + 2 < nsteps)
        def _():
            slab_copy(g0 + 2, slab_a, psem_a).start()
        cp_b.wait()
        extract(g0 + 1, slab_b)
        return carry
    lax.fori_loop(0, nsteps // 2, pair, 0)

    pltpu.sync_copy(sdata, s_out_hbm.at[pl.ds(wid * K_NN * t_len, K_NN * t_len)])


def _sc_gather_call(idx32, nbrw, yt, b, t_len, v_real):
    bpw = b // NW
    nrows = K_NN * t_len
    vq = v_real // NQ
    mesh = plsc.VectorSubcoreMesh(core_axis_name="c", subcore_axis_name="s")
    kern = pl.kernel(
        functools.partial(
            _sc_gather_body, bpw=bpw, t_len=t_len, v_real=v_real, vq=vq),
        out_type=(
            jax.ShapeDtypeStruct((NW * nrows, bpw), jnp.float32),
            jax.ShapeDtypeStruct((NW, 16, bpw), jnp.int32),
        ),
        mesh=mesh,
        compiler_params=pltpu.CompilerParams(
            use_tc_tiling_on_sc=True, needs_layout_passes=False),
        scratch_types=(
            pltpu.VMEM((bpw,), jnp.int32),
            pltpu.VMEM((bpw, 128), jnp.int32),
            pltpu.VMEM((16, bpw), jnp.int32),
            pltpu.VMEM((16, bpw), jnp.int32),
            pltpu.VMEM((NQ * K_NN, bpw), jnp.int32),
            pltpu.VMEM((NQ * K_NN, bpw), jnp.int32),
            pltpu.VMEM((vq, bpw), jnp.float32),
            pltpu.VMEM((vq, bpw), jnp.float32),
            pltpu.VMEM((nrows, bpw), jnp.float32),
            pltpu.SemaphoreType.DMA,
            pltpu.SemaphoreType.DMA,
            pltpu.SemaphoreType.DMA,
        ),
    )
    return kern(idx32, nbrw, yt)


def _combine_body(s_ref, w_ref, out_ref, *, t_len):
    s = s_ref[...]
    s3 = s.reshape(K_NN, t_len, s.shape[-1])
    wf = lax.bitcast_convert_type(w_ref[0], jnp.float32)
    w3 = wf[:K_NN][:, None, :]
    tw = jnp.where(s3 == 0.0, w3 / 1e9, w3)
    f1 = jnp.sum(tw, axis=0)
    f2 = jnp.sum(tw * s3, axis=0) / f1
    mean = jnp.sum(s3, axis=0) / K_NN
    var = jnp.sum((s3 - mean) ** 2, axis=0) / (K_NN - 1)
    f3 = jnp.sqrt(var)
    out_ref[...] = jnp.stack([f1, f2, f3], axis=0)


def _combine_call(s_out, wb, b, t_len):
    bpw = b // NW
    nrows = K_NN * t_len
    return pl.pallas_call(
        functools.partial(_combine_body, t_len=t_len),
        grid=(NW,),
        in_specs=[
            pl.BlockSpec((nrows, bpw), lambda i: (i, 0)),
            pl.BlockSpec((1, 16, bpw), lambda i: (i, 0, 0)),
        ],
        out_specs=pl.BlockSpec((3, t_len, bpw), lambda i: (0, 0, i)),
        out_shape=jax.ShapeDtypeStruct((3, t_len, b), jnp.float32),
    )(s_out, wb)


def kernel(embed_weight, y_context, indices):
    v_real, d = embed_weight.shape
    b, t_len, _ = y_context.shape
    vp = 1024

    wp = jnp.zeros((vp, 128), jnp.float32).at[:v_real, :d].set(embed_weight)
    wpt = wp.T

    nbrw = _topk_call(wp, wpt, v_real)

    idx32 = indices.astype(jnp.int32)
    yt = jnp.transpose(y_context, (1, 2, 0))
    s_out, wb = _sc_gather_call(idx32, nbrw, yt, b, t_len, v_real)
    out_c = _combine_call(s_out, wb, b, t_len)
    return jnp.transpose(out_c, (2, 1, 0))


# speedup vs baseline: 9.2158x; 1.0718x over previous
import functools

import jax
import jax.numpy as jnp
from jax import lax
from jax.experimental import pallas as pl
from jax.experimental.pallas import tpu as pltpu
from jax.experimental.pallas import tpu_sc as plsc

K_NN = 10
TAU = 1.0

NC, NS, LANES = 2, 16, 16
NW = NC * NS
NQ = 5


def _topk_body(wp_ref, wpt_ref, nbrw_ref, *, v_real):
    wp = wp_ref[...]
    wpt = wpt_ref[...]
    g = jnp.dot(wp, wpt, preferred_element_type=jnp.float32)
    n_row = jnp.sum(wp * wp, axis=1, keepdims=True)
    n_col = jnp.sum(wpt * wpt, axis=0, keepdims=True)
    d2 = n_row - 2.0 * g + n_col

    vp = d2.shape[0]
    col = lax.broadcasted_iota(jnp.int32, (vp, vp), 1)
    row = lax.broadcasted_iota(jnp.int32, (vp, vp), 0)
    inf = jnp.float32(3e38)
    d2m = jnp.where((col == row) | (col >= v_real), inf, d2)

    ids = []
    wbits = []
    for _ in range(K_NN):
        m = jnp.min(d2m, axis=1, keepdims=True)
        hit = d2m == m
        idx = jnp.min(jnp.where(hit, col, vp), axis=1, keepdims=True)
        ids.append(idx)
        w = jnp.exp(-(jnp.sqrt(jnp.maximum(m, 1e-12)) + 0.001) / TAU)
        wbits.append(lax.bitcast_convert_type(w, jnp.int32))
        d2m = jnp.where(col == idx, inf, d2m)

    zed = jnp.zeros((vp, 1), jnp.int32)
    cols = ids + [zed] * (16 - K_NN) + wbits + [zed] * (112 - K_NN)
    nbrw_ref[...] = jnp.concatenate(cols, axis=1)


def _topk_call(wp, wpt, v_real):
    vp = wp.shape[0]
    return pl.pallas_call(
        functools.partial(_topk_body, v_real=v_real),
        out_shape=jax.ShapeDtypeStruct((vp, 128), jnp.int32),
    )(wp, wpt)


def _sc_gather_body(idx_hbm, nbrw_hbm, yt_hbm, s_out_hbm, wb_out_hbm,
                    idxv, nwr, nbrt, wbt, vloc_pre,
                    slab_a, slab_b, slab_c, sdata, sem, psem_a, psem_b, psem_c,
                    *, bpw, t_len, v_real, vq):
    wid = lax.axis_index("s") * NC + lax.axis_index("c")
    base = wid * bpw
    bgs = bpw // LANES
    lane = lax.broadcasted_iota(jnp.int32, (LANES,), 0)
    nsteps = t_len * NQ
    bufs = ((slab_a, psem_a), (slab_b, psem_b), (slab_c, psem_c))

    def slab_copy(g, slab, psem):
        t = g // NQ
        h = g - t * NQ
        return pltpu.make_async_copy(
            yt_hbm.at[t, pl.ds(h * vq, vq), pl.ds(base, bpw)], slab, psem)

    for r, (slab, psem) in enumerate(bufs):
        slab_copy(r, slab, psem).start()

    pltpu.sync_copy(idx_hbm.at[pl.ds(base, bpw)], idxv)
    pltpu.async_copy(nbrw_hbm.at[idxv], nwr, sem).wait()

    for j in range(K_NN):
        jid = jnp.full((LANES,), j, jnp.int32)
        jwt = jnp.full((LANES,), 16 + j, jnp.int32)
        for bg in range(bgs):
            rows = bg * LANES + lane
            nbrt[j, pl.ds(bg * LANES, LANES)] = plsc.load_gather(nwr, [rows, jid])
            wbt[j, pl.ds(bg * LANES, LANES)] = plsc.load_gather(nwr, [rows, jwt])
    pltpu.sync_copy(wbt, wb_out_hbm.at[wid])

    for j in range(K_NN):
        for bg in range(bgs):
            sl = pl.ds(bg * LANES, LANES)
            v = nbrt[j, sl]
            for h in range(NQ):
                inr = (v >= h * vq) & (v < (h + 1) * vq)
                vloc_pre[h * K_NN + j, sl] = jnp.where(inr, v - h * vq, -1)

    def extract(g, slab):
        t = g // NQ
        h = g - t * NQ
        for j in range(K_NN):
            rowv = jnp.full((LANES,), j * t_len + t, jnp.int32)
            for bg in range(bgs):
                sl = pl.ds(bg * LANES, LANES)
                vl0 = vloc_pre[h * K_NN + j, sl]
                mk = vl0 >= 0
                vl = jnp.maximum(vl0, 0)
                bvec = bg * LANES + lane
                e = plsc.load_gather(slab, [vl, bvec])
                plsc.store_scatter(sdata, [rowv, bvec], e, mask=mk)

    def triple(p, carry):
        g0 = 3 * p
        for r, (slab, psem) in enumerate(bufs):
            g = g0 + r
            slab_copy(g, slab, psem).wait()
            extract(g, slab)
            @pl.when(g + 3 < nsteps)
            def _(g=g, slab=slab, psem=psem):
                slab_copy(g + 3, slab, psem).start()
        return carry
    lax.fori_loop(0, nsteps // 3, triple, 0)
    for g in range(nsteps - (nsteps % 3), nsteps):
        slab, psem = bufs[g % 3]
        slab_copy(g, slab, psem).wait()
        extract(g, slab)

    pltpu.sync_copy(sdata, s_out_hbm.at[pl.ds(wid * K_NN * t_len, K_NN * t_len)])


def _sc_gather_call(idx32, nbrw, yt, b, t_len, v_real):
    bpw = b // NW
    nrows = K_NN * t_len
    vq = v_real // NQ
    mesh = plsc.VectorSubcoreMesh(core_axis_name="c", subcore_axis_name="s")
    kern = pl.kernel(
        functools.partial(
            _sc_gather_body, bpw=bpw, t_len=t_len, v_real=v_real, vq=vq),
        out_type=(
            jax.ShapeDtypeStruct((NW * nrows, bpw), jnp.float32),
            jax.ShapeDtypeStruct((NW, 16, bpw), jnp.int32),
        ),
        mesh=mesh,
        compiler_params=pltpu.CompilerParams(
            use_tc_tiling_on_sc=True, needs_layout_passes=False),
        scratch_types=(
            pltpu.VMEM((bpw,), jnp.int32),
            pltpu.VMEM((bpw, 128), jnp.int32),
            pltpu.VMEM((16, bpw), jnp.int32),
            pltpu.VMEM((16, bpw), jnp.int32),
            pltpu.VMEM((NQ * K_NN, bpw), jnp.int32),
            pltpu.VMEM((vq, bpw), jnp.float32),
            pltpu.VMEM((vq, bpw), jnp.float32),
            pltpu.VMEM((vq, bpw), jnp.float32),
            pltpu.VMEM((nrows, bpw), jnp.float32),
            pltpu.SemaphoreType.DMA,
            pltpu.SemaphoreType.DMA,
            pltpu.SemaphoreType.DMA,
            pltpu.SemaphoreType.DMA,
        ),
    )
    return kern(idx32, nbrw, yt)


def _combine_body(s_ref, w_ref, out_ref, *, t_len):
    s = s_ref[...]
    s3 = s.reshape(K_NN, t_len, s.shape[-1])
    wf = lax.bitcast_convert_type(w_ref[0], jnp.float32)
    w3 = wf[:K_NN][:, None, :]
    tw = jnp.where(s3 == 0.0, w3 / 1e9, w3)
    f1 = jnp.sum(tw, axis=0)
    f2 = jnp.sum(tw * s3, axis=0) / f1
    mean = jnp.sum(s3, axis=0) / K_NN
    var = jnp.sum((s3 - mean) ** 2, axis=0) / (K_NN - 1)
    f3 = jnp.sqrt(var)
    out_ref[...] = jnp.stack([f1, f2, f3], axis=0)


def _combine_call(s_out, wb, b, t_len):
    bpw = b // NW
    nrows = K_NN * t_len
    return pl.pallas_call(
        functools.partial(_combine_body, t_len=t_len),
        grid=(NW,),
        in_specs=[
            pl.BlockSpec((nrows, bpw), lambda i: (i, 0)),
            pl.BlockSpec((1, 16, bpw), lambda i: (i, 0, 0)),
        ],
        out_specs=pl.BlockSpec((3, t_len, bpw), lambda i: (0, 0, i)),
        out_shape=jax.ShapeDtypeStruct((3, t_len, b), jnp.float32),
    )(s_out, wb)


def kernel(embed_weight, y_context, indices):
    v_real, d = embed_weight.shape
    b, t_len, _ = y_context.shape
    vp = 1024

    wp = jnp.zeros((vp, 128), jnp.float32).at[:v_real, :d].set(embed_weight)
    wpt = wp.T

    nbrw = _topk_call(wp, wpt, v_real)

    idx32 = indices.astype(jnp.int32)
    yt = jnp.transpose(y_context, (1, 2, 0))
    s_out, wb = _sc_gather_call(idx32, nbrw, yt, b, t_len, v_real)
    out_c = _combine_call(s_out, wb, b, t_len)
    return jnp.transpose(out_c, (2, 1, 0))


# speedup vs baseline: 9.7352x; 1.0564x over previous
import functools

import jax
import jax.numpy as jnp
from jax import lax
from jax.experimental import pallas as pl
from jax.experimental.pallas import tpu as pltpu
from jax.experimental.pallas import tpu_sc as plsc

K_NN = 10
TAU = 1.0

NC, NS, LANES = 2, 16, 16
NW = NC * NS
NQ = 5


def _topk_body(wp_ref, wpt_ref, nbrw_ref, *, v_real):
    wp = wp_ref[...]
    wpt = wpt_ref[...]
    g = jnp.dot(wp, wpt, preferred_element_type=jnp.float32)
    n_row = jnp.sum(wp * wp, axis=1, keepdims=True)
    n_col = jnp.sum(wpt * wpt, axis=0, keepdims=True)
    d2 = n_row - 2.0 * g + n_col

    vp = d2.shape[0]
    col = lax.broadcasted_iota(jnp.int32, (vp, vp), 1)
    row = lax.broadcasted_iota(jnp.int32, (vp, vp), 0)
    inf = jnp.float32(3e38)
    d2m = jnp.where((col == row) | (col >= v_real), inf, d2)

    ids = []
    wbits = []
    for _ in range(K_NN):
        m = jnp.min(d2m, axis=1, keepdims=True)
        hit = d2m == m
        idx = jnp.min(jnp.where(hit, col, vp), axis=1, keepdims=True)
        ids.append(idx)
        w = jnp.exp(-(jnp.sqrt(jnp.maximum(m, 1e-12)) + 0.001) / TAU)
        wbits.append(lax.bitcast_convert_type(w, jnp.int32))
        d2m = jnp.where(col == idx, inf, d2m)

    zed = jnp.zeros((vp, 1), jnp.int32)
    cols = ids + [zed] * (16 - K_NN) + wbits + [zed] * (112 - K_NN)
    nbrw_ref[...] = jnp.concatenate(cols, axis=1)


def _topk_call(wp, wpt, v_real):
    vp = wp.shape[0]
    return pl.pallas_call(
        functools.partial(_topk_body, v_real=v_real),
        out_shape=jax.ShapeDtypeStruct((vp, 128), jnp.int32),
    )(wp, wpt)


def _sc_gather_body(idx_hbm, nbrw_hbm, yt_hbm, s_out_hbm, wb_out_hbm,
                    idxv, nwr, nbrt, wbt, vloc_pre,
                    slab_a, slab_b, slab_c, sdata, sem, psem_a, psem_b, psem_c,
                    *, bpw, t_len, v_real, vq):
    wid = lax.axis_index("s") * NC + lax.axis_index("c")
    base = wid * bpw
    bgs = bpw // LANES
    lane = lax.broadcasted_iota(jnp.int32, (LANES,), 0)
    nsteps = t_len * NQ
    bufs = ((slab_a, psem_a), (slab_b, psem_b), (slab_c, psem_c))

    def slab_copy(g, slab, psem):
        t = g // NQ
        h = g - t * NQ
        return pltpu.make_async_copy(
            yt_hbm.at[t, pl.ds(h * vq, vq), pl.ds(base, bpw)], slab, psem)

    for r, (slab, psem) in enumerate(bufs):
        slab_copy(r, slab, psem).start()

    pltpu.sync_copy(idx_hbm.at[pl.ds(base, bpw)], idxv)
    pltpu.async_copy(nbrw_hbm.at[idxv], nwr, sem).wait()

    for j in range(K_NN):
        jid = jnp.full((LANES,), j, jnp.int32)
        jwt = jnp.full((LANES,), 16 + j, jnp.int32)
        for bg in range(bgs):
            rows = bg * LANES + lane
            nbrt[j, pl.ds(bg * LANES, LANES)] = plsc.load_gather(nwr, [rows, jid])
            wbt[j, pl.ds(bg * LANES, LANES)] = plsc.load_gather(nwr, [rows, jwt])
    pltpu.sync_copy(wbt, wb_out_hbm.at[wid])

    for j in range(K_NN):
        for bg in range(bgs):
            sl = pl.ds(bg * LANES, LANES)
            v = nbrt[j, sl]
            for h in range(NQ):
                inr = (v >= h * vq) & (v < (h + 1) * vq)
                vloc_pre[h * K_NN + j, sl] = jnp.where(inr, v - h * vq, -1)

    def extract(g, slab):
        t = g // NQ
        h = g - t * NQ
        for j in range(K_NN):
            rowv = jnp.full((LANES,), j * t_len + t, jnp.int32)
            for bg in range(bgs):
                sl = pl.ds(bg * LANES, LANES)
                vl0 = vloc_pre[h * K_NN + j, sl]
                mk = vl0 >= 0
                vl = jnp.maximum(vl0, 0)
                bvec = bg * LANES + lane
                e = plsc.load_gather(slab, [vl, bvec])
                plsc.store_scatter(sdata, [rowv, bvec], e, mask=mk)

    def triple(p, carry):
        g0 = 3 * p
        for r, (slab, psem) in enumerate(bufs):
            g = g0 + r
            slab_copy(g, slab, psem).wait()
            extract(g, slab)
            @pl.when(g + 3 < nsteps)
            def _(g=g, slab=slab, psem=psem):
                slab_copy(g + 3, slab, psem).start()
        return carry
    lax.fori_loop(0, nsteps // 3, triple, 0)
    for g in range(nsteps - (nsteps % 3), nsteps):
        slab, psem = bufs[g % 3]
        slab_copy(g, slab, psem).wait()
        extract(g, slab)

    pltpu.sync_copy(sdata, s_out_hbm.at[pl.ds(wid * K_NN * t_len, K_NN * t_len)])


def _sc_gather_call(idx32, nbrw, yt, b, t_len, v_real):
    bpw = b // NW
    nrows = K_NN * t_len
    vq = v_real // NQ
    mesh = plsc.VectorSubcoreMesh(core_axis_name="c", subcore_axis_name="s")
    kern = pl.kernel(
        functools.partial(
            _sc_gather_body, bpw=bpw, t_len=t_len, v_real=v_real, vq=vq),
        out_type=(
            jax.ShapeDtypeStruct((NW * nrows, bpw), jnp.float32),
            jax.ShapeDtypeStruct((NW, 16, bpw), jnp.int32),
        ),
        mesh=mesh,
        compiler_params=pltpu.CompilerParams(
            use_tc_tiling_on_sc=True, needs_layout_passes=False),
        scratch_types=(
            pltpu.VMEM((bpw,), jnp.int32),
            pltpu.VMEM((bpw, 128), jnp.int32),
            pltpu.VMEM((16, bpw), jnp.int32),
            pltpu.VMEM((16, bpw), jnp.int32),
            pltpu.VMEM((NQ * K_NN, bpw), jnp.int32),
            pltpu.VMEM((vq, bpw), jnp.float32),
            pltpu.VMEM((vq, bpw), jnp.float32),
            pltpu.VMEM((vq, bpw), jnp.float32),
            pltpu.VMEM((nrows, bpw), jnp.float32),
            pltpu.SemaphoreType.DMA,
            pltpu.SemaphoreType.DMA,
            pltpu.SemaphoreType.DMA,
            pltpu.SemaphoreType.DMA,
        ),
    )
    return kern(idx32, nbrw, yt)


def _combine_body(s_ref, w_ref, out_ref, *, t_len, wpb):
    nrows = K_NN * t_len
    f1s, f2s, f3s = [], [], []
    for c in range(wpb):
        s3 = s_ref[c * nrows:(c + 1) * nrows, :].reshape(K_NN, t_len, -1)
        wf = lax.bitcast_convert_type(w_ref[c], jnp.float32)
        w3 = wf[:K_NN][:, None, :]
        tw = jnp.where(s3 == 0.0, w3 / 1e9, w3)
        f1 = jnp.sum(tw, axis=0)
        f1s.append(f1)
        f2s.append(jnp.sum(tw * s3, axis=0) / f1)
        mean = jnp.sum(s3, axis=0) / K_NN
        var = jnp.sum((s3 - mean) ** 2, axis=0) / (K_NN - 1)
        f3s.append(jnp.sqrt(var))
    out_ref[...] = jnp.stack(
        [jnp.concatenate(f1s, axis=1),
         jnp.concatenate(f2s, axis=1),
         jnp.concatenate(f3s, axis=1)], axis=0)


def _combine_call(s_out, wb, b, t_len):
    bpw = b // NW
    nrows = K_NN * t_len
    wpb = 4
    return pl.pallas_call(
        functools.partial(_combine_body, t_len=t_len, wpb=wpb),
        grid=(NW // wpb,),
        in_specs=[
            pl.BlockSpec((wpb * nrows, bpw), lambda i: (i, 0)),
            pl.BlockSpec((wpb, 16, bpw), lambda i: (i, 0, 0)),
        ],
        out_specs=pl.BlockSpec((3, t_len, wpb * bpw), lambda i: (0, 0, i)),
        out_shape=jax.ShapeDtypeStruct((3, t_len, b), jnp.float32),
    )(s_out, wb)


def kernel(embed_weight, y_context, indices):
    v_real, d = embed_weight.shape
    b, t_len, _ = y_context.shape
    vp = 1024

    wp = jnp.zeros((vp, 128), jnp.float32).at[:v_real, :d].set(embed_weight)
    wpt = wp.T

    nbrw = _topk_call(wp, wpt, v_real)

    idx32 = indices.astype(jnp.int32)
    yt = jnp.transpose(y_context, (1, 2, 0))
    s_out, wb = _sc_gather_call(idx32, nbrw, yt, b, t_len, v_real)
    out_c = _combine_call(s_out, wb, b, t_len)
    return jnp.transpose(out_c, (2, 1, 0))


# speedup vs baseline: 10.9325x; 1.1230x over previous
import functools

import jax
import jax.numpy as jnp
from jax import lax
from jax.experimental import pallas as pl
from jax.experimental.pallas import tpu as pltpu
from jax.experimental.pallas import tpu_sc as plsc

K_NN = 10
TAU = 1.0

NC, NS, LANES = 2, 16, 16
NW = NC * NS
NQ = 5


def _topk_body(wp_ref, wpt_ref, nbrw_ref, *, v_real):
    wp = wp_ref[...]
    wpt = wpt_ref[...]
    g = jnp.dot(wp, wpt, preferred_element_type=jnp.float32)
    n_row = jnp.sum(wp * wp, axis=1, keepdims=True)
    n_col = jnp.sum(wpt * wpt, axis=0, keepdims=True)
    d2 = n_row - 2.0 * g + n_col

    vp = d2.shape[0]
    col = lax.broadcasted_iota(jnp.int32, (vp, vp), 1)
    row = lax.broadcasted_iota(jnp.int32, (vp, vp), 0)
    inf = jnp.float32(3e38)
    d2m = jnp.where((col == row) | (col >= v_real), inf, d2)

    ids = []
    wbits = []
    for _ in range(K_NN):
        m = jnp.min(d2m, axis=1, keepdims=True)
        hit = d2m == m
        idx = jnp.min(jnp.where(hit, col, vp), axis=1, keepdims=True)
        ids.append(idx)
        w = jnp.exp(-(jnp.sqrt(jnp.maximum(m, 1e-12)) + 0.001) / TAU)
        wbits.append(lax.bitcast_convert_type(w, jnp.int32))
        d2m = jnp.where(col == idx, inf, d2m)

    zed = jnp.zeros((vp, 1), jnp.int32)
    cols = ids + [zed] * (16 - K_NN) + wbits + [zed] * (112 - K_NN)
    nbrw_ref[...] = jnp.concatenate(cols, axis=1)


def _topk_call(wp, wpt, v_real):
    vp = wp.shape[0]
    return pl.pallas_call(
        functools.partial(_topk_body, v_real=v_real),
        out_shape=jax.ShapeDtypeStruct((vp, 128), jnp.int32),
    )(wp, wpt)


def _sc_gather_body(idx_hbm, nbrw_hbm, yt_hbm, s_out_hbm, wb_out_hbm,
                    idxv, nwr, nbrt, wbt, vloc_pre,
                    slab_a, slab_b, slab_c, sdata, sem, psem_a, psem_b, psem_c,
                    *, bpw, t_len, v_real, vq):
    wid = lax.axis_index("s") * NC + lax.axis_index("c")
    base = wid * bpw
    bgs = bpw // LANES
    lane = lax.broadcasted_iota(jnp.int32, (LANES,), 0)
    nsteps = t_len * NQ
    bufs = ((slab_a, psem_a), (slab_b, psem_b), (slab_c, psem_c))

    def slab_copy(g, slab, psem):
        t = g // NQ
        h = g - t * NQ
        return pltpu.make_async_copy(
            yt_hbm.at[t, pl.ds(h * vq, vq), pl.ds(base, bpw)], slab, psem)

    for r, (slab, psem) in enumerate(bufs):
        slab_copy(r, slab, psem).start()

    pltpu.sync_copy(idx_hbm.at[pl.ds(base, bpw)], idxv)
    pltpu.async_copy(nbrw_hbm.at[idxv], nwr, sem).wait()

    for j in range(K_NN):
        jid = jnp.full((LANES,), j, jnp.int32)
        jwt = jnp.full((LANES,), 16 + j, jnp.int32)
        for bg in range(bgs):
            rows = bg * LANES + lane
            nbrt[j, pl.ds(bg * LANES, LANES)] = plsc.load_gather(nwr, [rows, jid])
            wbt[j, pl.ds(bg * LANES, LANES)] = plsc.load_gather(nwr, [rows, jwt])
    pltpu.sync_copy(wbt, wb_out_hbm.at[wid])

    for j in range(K_NN):
        for bg in range(bgs):
            sl = pl.ds(bg * LANES, LANES)
            v = nbrt[j, sl]
            for h in range(NQ):
                inr = (v >= h * vq) & (v < (h + 1) * vq)
                vloc_pre[h * K_NN + j, sl] = jnp.where(inr, v - h * vq, -1)

    def extract(g, slab):
        t = g // NQ
        h = g - t * NQ
        for j in range(K_NN):
            rowv = jnp.full((LANES,), j * t_len + t, jnp.int32)
            for bg in range(bgs):
                sl = pl.ds(bg * LANES, LANES)
                vl0 = vloc_pre[h * K_NN + j, sl]
                mk = vl0 >= 0
                bvec = bg * LANES + lane
                e = plsc.load_gather(slab, [vl0, bvec], mask=mk)
                plsc.store_scatter(sdata, [rowv, bvec], e, mask=mk)

    def triple(p, carry):
        g0 = 3 * p
        for r, (slab, psem) in enumerate(bufs):
            g = g0 + r
            slab_copy(g, slab, psem).wait()
            extract(g, slab)
            @pl.when(g + 3 < nsteps)
            def _(g=g, slab=slab, psem=psem):
                slab_copy(g + 3, slab, psem).start()
        return carry
    lax.fori_loop(0, nsteps // 3, triple, 0)
    for g in range(nsteps - (nsteps % 3), nsteps):
        slab, psem = bufs[g % 3]
        slab_copy(g, slab, psem).wait()
        extract(g, slab)

    pltpu.sync_copy(sdata, s_out_hbm.at[pl.ds(wid * K_NN * t_len, K_NN * t_len)])


def _sc_gather_call(idx32, nbrw, yt, b, t_len, v_real):
    bpw = b // NW
    nrows = K_NN * t_len
    vq = v_real // NQ
    mesh = plsc.VectorSubcoreMesh(core_axis_name="c", subcore_axis_name="s")
    kern = pl.kernel(
        functools.partial(
            _sc_gather_body, bpw=bpw, t_len=t_len, v_real=v_real, vq=vq),
        out_type=(
            jax.ShapeDtypeStruct((NW * nrows, bpw), jnp.float32),
            jax.ShapeDtypeStruct((NW, 16, bpw), jnp.int32),
        ),
        mesh=mesh,
        compiler_params=pltpu.CompilerParams(
            use_tc_tiling_on_sc=True, needs_layout_passes=False),
        scratch_types=(
            pltpu.VMEM((bpw,), jnp.int32),
            pltpu.VMEM((bpw, 128), jnp.int32),
            pltpu.VMEM((16, bpw), jnp.int32),
            pltpu.VMEM((16, bpw), jnp.int32),
            pltpu.VMEM((NQ * K_NN, bpw), jnp.int32),
            pltpu.VMEM((vq, bpw), jnp.float32),
            pltpu.VMEM((vq, bpw), jnp.float32),
            pltpu.VMEM((vq, bpw), jnp.float32),
            pltpu.VMEM((nrows, bpw), jnp.float32),
            pltpu.SemaphoreType.DMA,
            pltpu.SemaphoreType.DMA,
            pltpu.SemaphoreType.DMA,
            pltpu.SemaphoreType.DMA,
        ),
    )
    return kern(idx32, nbrw, yt)


def _combine_body(s_ref, w_ref, out_ref, *, t_len, wpb):
    nrows = K_NN * t_len
    f1s, f2s, f3s = [], [], []
    for c in range(wpb):
        s3 = s_ref[c * nrows:(c + 1) * nrows, :].reshape(K_NN, t_len, -1)
        wf = lax.bitcast_convert_type(w_ref[c], jnp.float32)
        w3 = wf[:K_NN][:, None, :]
        tw = jnp.where(s3 == 0.0, w3 / 1e9, w3)
        f1 = jnp.sum(tw, axis=0)
        f1s.append(f1)
        f2s.append(jnp.sum(tw * s3, axis=0) / f1)
        mean = jnp.sum(s3, axis=0) / K_NN
        var = jnp.sum((s3 - mean) ** 2, axis=0) / (K_NN - 1)
        f3s.append(jnp.sqrt(var))
    out_ref[...] = jnp.stack(
        [jnp.concatenate(f1s, axis=1),
         jnp.concatenate(f2s, axis=1),
         jnp.concatenate(f3s, axis=1)], axis=0)


def _combine_call(s_out, wb, b, t_len):
    bpw = b // NW
    nrows = K_NN * t_len
    wpb = 4
    return pl.pallas_call(
        functools.partial(_combine_body, t_len=t_len, wpb=wpb),
        grid=(NW // wpb,),
        in_specs=[
            pl.BlockSpec((wpb * nrows, bpw), lambda i: (i, 0)),
            pl.BlockSpec((wpb, 16, bpw), lambda i: (i, 0, 0)),
        ],
        out_specs=pl.BlockSpec((3, t_len, wpb * bpw), lambda i: (0, 0, i)),
        out_shape=jax.ShapeDtypeStruct((3, t_len, b), jnp.float32),
    )(s_out, wb)


def kernel(embed_weight, y_context, indices):
    v_real, d = embed_weight.shape
    b, t_len, _ = y_context.shape
    vp = 1024

    wp = jnp.zeros((vp, 128), jnp.float32).at[:v_real, :d].set(embed_weight)
    wpt = wp.T

    nbrw = _topk_call(wp, wpt, v_real)

    idx32 = indices.astype(jnp.int32)
    yt = jnp.transpose(y_context, (1, 2, 0))
    s_out, wb = _sc_gather_call(idx32, nbrw, yt, b, t_len, v_real)
    out_c = _combine_call(s_out, wb, b, t_len)
    return jnp.transpose(out_c, (2, 1, 0))
